# baseline (speedup 1.0000x reference)
import functools

import jax
import jax.numpy as jnp
from jax import lax
from jax.experimental import pallas as pl
from jax.experimental.pallas import tpu as pltpu
from jax.experimental.pallas import tpu_sc as plsc

EMBED_D = 64
SLAB_B = 128
SCH = 4


def _sc_gather(idx_t, table_flat):
    n_seq, n_batch = idx_t.shape
    n_table = table_flat.shape[0]
    info = plsc.get_sparse_core_info()
    nc, ns = info.num_cores, info.num_subcores
    nw = nc * ns
    slabs_per_w = n_batch // SLAB_B // nw
    n_chunks = n_seq // SCH
    n_pairs = n_chunks // 2

    mesh = plsc.VectorSubcoreMesh(core_axis_name="c", subcore_axis_name="s")

    @functools.partial(
        pl.kernel,
        mesh=mesh,
        out_type=jax.ShapeDtypeStruct((n_seq, EMBED_D, n_batch), jnp.float32),
        scratch_types=[
            pltpu.VMEM((n_table,), jnp.float32),
            pltpu.VMEM((n_seq, SLAB_B), jnp.int32),
            pltpu.VMEM((SCH, EMBED_D, SLAB_B), jnp.float32),
            pltpu.VMEM((SCH, EMBED_D, SLAB_B), jnp.float32),
            pltpu.SemaphoreType.DMA,
            pltpu.SemaphoreType.DMA,
        ],
        compiler_params=pltpu.CompilerParams(
            use_tc_tiling_on_sc=True,
            needs_layout_passes=False,
            disable_bounds_checks=True,
        ),
    )
    def k(table_hbm, idx_hbm, out_hbm, table_v, idx_sl, rows_v0, rows_v1, sem0, sem1):
        pltpu.sync_copy(table_hbm, table_v)
        wid = lax.axis_index("s") * nc + lax.axis_index("c")
        lane = lax.iota(jnp.int32, 16)

        def process(g, s0, bg, slot_rows, sem):
            @pl.when(g >= 2)
            def _():
                pltpu.make_async_copy(
                    slot_rows,
                    out_hbm.at[pl.ds(0, SCH), :, pl.ds(0, SLAB_B)],
                    sem,
                ).wait()

            @plsc.parallel_loop(0, SCH * (SLAB_B // 16), unroll=5)
            def group(t):
                s_l = t // (SLAB_B // 16)
                b0 = (t % (SLAB_B // 16)) * 16
                b_vec = b0 + lane
                r_vec = idx_sl[s0 + s_l, pl.ds(b0, 16)]
                src_base = r_vec * EMBED_D
                sl_vec = jnp.full((16,), s_l, jnp.int32)
                for j in range(EMBED_D):
                    c = lane ^ j
                    w = plsc.load_gather(table_v, [src_base + c])
                    plsc.store_scatter(slot_rows, [sl_vec, c, b_vec], w)

            pltpu.async_copy(
                slot_rows,
                out_hbm.at[pl.ds(s0, SCH), :, pl.ds(bg, SLAB_B)],
                sem,
            )

        def slab_body(kslab, carry):
            bg = (wid * slabs_per_w + kslab) * SLAB_B
            pltpu.sync_copy(idx_hbm.at[:, pl.ds(bg, SLAB_B)], idx_sl)

            def pair(p, c2):
                g = kslab * n_chunks + 2 * p
                process(g, (2 * p) * SCH, bg, rows_v0, sem0)
                process(g + 1, (2 * p + 1) * SCH, bg, rows_v1, sem1)
                return c2

            lax.fori_loop(0, n_pairs, pair, 0)
            return carry

        lax.fori_loop(0, slabs_per_w, slab_body, 0)

        pltpu.make_async_copy(
            rows_v0, out_hbm.at[pl.ds(0, SCH), :, pl.ds(0, SLAB_B)], sem0
        ).wait()
        pltpu.make_async_copy(
            rows_v1, out_hbm.at[pl.ds(0, SCH), :, pl.ds(0, SLAB_B)], sem1
        ).wait()

    return k(table_flat, idx_t)


def kernel(time_idx, table):
    out_t = _sc_gather(time_idx.T, table.reshape(-1))
    return jnp.transpose(out_t, (2, 0, 1))


# speedup vs baseline: 1.3416x; 1.3416x over previous
import functools

import jax
import jax.numpy as jnp
from jax import lax
from jax.experimental import pallas as pl
from jax.experimental.pallas import tpu as pltpu
from jax.experimental.pallas import tpu_sc as plsc

EMBED_D = 64
SLAB_B = 128
SCH = 4


def _sc_gather(idx_t, table_flat):
    n_seq, n_batch = idx_t.shape
    n_table = table_flat.shape[0]
    info = plsc.get_sparse_core_info()
    nc, ns = info.num_cores, info.num_subcores
    nw = nc * ns
    slabs_per_w = n_batch // SLAB_B // nw
    n_chunks = n_seq // SCH
    n_pairs = n_chunks // 2

    mesh = plsc.VectorSubcoreMesh(core_axis_name="c", subcore_axis_name="s")

    @functools.partial(
        pl.kernel,
        mesh=mesh,
        out_type=jax.ShapeDtypeStruct((n_seq, EMBED_D, n_batch), jnp.float32),
        scratch_types=[
            pltpu.VMEM((n_table,), jnp.float32),
            pltpu.VMEM((n_seq, SLAB_B), jnp.int32),
            pltpu.VMEM((SCH, EMBED_D, SLAB_B), jnp.float32),
            pltpu.VMEM((SCH, EMBED_D, SLAB_B), jnp.float32),
            pltpu.SemaphoreType.DMA,
            pltpu.SemaphoreType.DMA,
        ],
        compiler_params=pltpu.CompilerParams(
            use_tc_tiling_on_sc=True,
            needs_layout_passes=False,
            disable_bounds_checks=True,
        ),
    )
    def k(table_hbm, idx_hbm, out_hbm, table_v, idx_sl, rows_v0, rows_v1, sem0, sem1):
        pltpu.sync_copy(table_hbm, table_v)
        wid = lax.axis_index("s") * nc + lax.axis_index("c")
        lane = lax.iota(jnp.int32, 16)

        def process(g, s0, bg, slot_rows, sem):
            @pl.when(g >= 2)
            def _():
                pltpu.make_async_copy(
                    slot_rows,
                    out_hbm.at[pl.ds(0, SCH), :, pl.ds(0, SLAB_B)],
                    sem,
                ).wait()

            @plsc.parallel_loop(0, SCH * (SLAB_B // 16), unroll=4)
            def group(t):
                s_l = t // (SLAB_B // 16)
                b0 = (t % (SLAB_B // 16)) * 16
                b_vec = b0 + lane
                r_vec = idx_sl[s0 + s_l, pl.ds(b0, 16)]
                src_base = r_vec * EMBED_D
                sl_vec = jnp.full((16,), s_l, jnp.int32)
                for j in range(EMBED_D):
                    c = lane ^ j
                    w = plsc.load_gather(table_v, [src_base + c])
                    plsc.store_scatter(slot_rows, [sl_vec, c, b_vec], w)

            pltpu.async_copy(
                slot_rows,
                out_hbm.at[pl.ds(s0, SCH), :, pl.ds(bg, SLAB_B)],
                sem,
            )

        def slab_body(kslab, carry):
            bg = (wid * slabs_per_w + kslab) * SLAB_B
            pltpu.sync_copy(idx_hbm.at[:, pl.ds(bg, SLAB_B)], idx_sl)

            def pair(p, c2):
                g = kslab * n_chunks + 2 * p
                process(g, (2 * p) * SCH, bg, rows_v0, sem0)
                process(g + 1, (2 * p + 1) * SCH, bg, rows_v1, sem1)
                return c2

            lax.fori_loop(0, n_pairs, pair, 0)
            return carry

        lax.fori_loop(0, slabs_per_w, slab_body, 0)

        pltpu.make_async_copy(
            rows_v0, out_hbm.at[pl.ds(0, SCH), :, pl.ds(0, SLAB_B)], sem0
        ).wait()
        pltpu.make_async_copy(
            rows_v1, out_hbm.at[pl.ds(0, SCH), :, pl.ds(0, SLAB_B)], sem1
        ).wait()

    return k(table_flat, idx_t)


def kernel(time_idx, table):
    out_t = _sc_gather(time_idx.T, table.reshape(-1))
    return jnp.transpose(out_t, (2, 0, 1))
